# Initial kernel scaffold; baseline (speedup 1.0000x reference)
import jax
import jax.numpy as jnp
from jax.experimental import pallas as pl


def kernel(idx, u):
    raise NotImplementedError("write your pallas kernel here")


# speedup vs baseline: 173.7259x; 173.7259x over previous
import jax
import jax.numpy as jnp
from jax import lax
from jax.experimental import pallas as pl
from jax.experimental.pallas import tpu as pltpu
from jax.experimental.pallas import tpu_sc as plsc

LENGTH = 1_000_000
N = 16384 * 200
NC, NS = 2, 16
NW = NC * NS
PER_W = N // NW
ROW = 128
WROWS = 16
W = ROW * WROWS
WINDOWS = PER_W // W
LOAD_CHUNK = 50_000
LOAD_CHUNKS = LENGTH // LOAD_CHUNK


def _body(idx_hbm, u_hbm, out_hbm, u_sp, bounce_v, idx_v, out_v, sem_g):
    c = lax.axis_index("c")
    s = lax.axis_index("s")
    wid = s * NC + c

    @pl.loop(s, LOAD_CHUNKS, step=NS)
    def _(ci):
        off = ci * LOAD_CHUNK
        pltpu.sync_copy(u_hbm.at[pl.ds(off, LOAD_CHUNK)], bounce_v)
        pltpu.sync_copy(bounce_v, u_sp.at[pl.ds(off, LOAD_CHUNK)])

    plsc.subcore_barrier()

    row0 = wid * (PER_W // ROW)

    @pl.loop(0, WINDOWS)
    def _(w):
        rbase = row0 + w * WROWS
        pltpu.sync_copy(idx_hbm.at[pl.ds(rbase, WROWS)], idx_v)
        cps = [pltpu.async_copy(u_sp.at[idx_v.at[j]], out_v.at[j], sem_g)
               for j in range(WROWS)]
        for cp in cps:
            cp.wait()
        pltpu.sync_copy(out_v, out_hbm.at[pl.ds(rbase, WROWS)])


def kernel(idx, u):
    idx2d = idx.reshape(N // ROW, ROW)
    out = pl.kernel(
        _body,
        out_type=jax.ShapeDtypeStruct((N // ROW, ROW), jnp.float32),
        mesh=plsc.VectorSubcoreMesh(core_axis_name="c", subcore_axis_name="s"),
        scratch_types=[
            pltpu.VMEM_SHARED((LENGTH,), jnp.float32),
            pltpu.VMEM((LOAD_CHUNK,), jnp.float32),
            pltpu.VMEM((WROWS, ROW), jnp.int32),
            pltpu.VMEM((WROWS, ROW), jnp.float32),
            pltpu.SemaphoreType.DMA,
        ],
    )(idx2d, u)
    return out.reshape(idx.shape)


# speedup vs baseline: 219.1270x; 1.2613x over previous
import jax
import jax.numpy as jnp
from jax import lax
from jax.experimental import pallas as pl
from jax.experimental.pallas import tpu as pltpu
from jax.experimental.pallas import tpu_sc as plsc

LENGTH = 1_000_000
N = 16384 * 200
NC, NS = 2, 16
NW = NC * NS
PER_W = N // NW
ROW = 128
WROWS = 16
W = ROW * WROWS
WINDOWS = PER_W // W
LOAD_CHUNK = 50_000
LOAD_CHUNKS = LENGTH // LOAD_CHUNK


def _body(idx_hbm, u_hbm, out_hbm, u_sp, bounce_v, idx_v, out_v,
          sem_g, sem_i0, sem_i1, sem_o0, sem_o1):
    c = lax.axis_index("c")
    s = lax.axis_index("s")
    wid = s * NC + c
    row0 = wid * (PER_W // ROW)
    sem_i = (sem_i0, sem_i1)
    sem_o = (sem_o0, sem_o1)

    def idx_rows(wi):
        return idx_hbm.at[pl.ds(row0 + wi * WROWS, WROWS)]

    def out_rows(wi):
        return out_hbm.at[pl.ds(row0 + wi * WROWS, WROWS)]

    pltpu.async_copy(idx_rows(0), idx_v.at[0], sem_i0)
    pltpu.async_copy(idx_rows(1), idx_v.at[1], sem_i1)

    @pl.loop(s, LOAD_CHUNKS, step=NS)
    def _(ci):
        off = ci * LOAD_CHUNK
        pltpu.sync_copy(u_hbm.at[pl.ds(off, LOAD_CHUNK)], bounce_v)
        pltpu.sync_copy(bounce_v, u_sp.at[pl.ds(off, LOAD_CHUNK)])

    plsc.subcore_barrier()

    @pl.loop(0, WINDOWS, step=2)
    def _(w):
        for b in range(2):
            wi = w + b
            pltpu.make_async_copy(idx_rows(wi), idx_v.at[b], sem_i[b]).wait()
            @pl.when(wi >= 2)
            def _():
                pltpu.make_async_copy(out_v.at[b], out_rows(wi - 2),
                                      sem_o[b]).wait()
            cps = [pltpu.async_copy(u_sp.at[idx_v.at[b].at[j]],
                                    out_v.at[b].at[j], sem_g)
                   for j in range(WROWS)]
            for cp in cps:
                cp.wait()
            @pl.when(wi + 2 < WINDOWS)
            def _():
                pltpu.async_copy(idx_rows(wi + 2), idx_v.at[b], sem_i[b])
            pltpu.async_copy(out_v.at[b], out_rows(wi), sem_o[b])

    pltpu.make_async_copy(out_v.at[0], out_rows(WINDOWS - 2), sem_o0).wait()
    pltpu.make_async_copy(out_v.at[1], out_rows(WINDOWS - 1), sem_o1).wait()


def kernel(idx, u):
    idx2d = idx.reshape(N // ROW, ROW)
    out = pl.kernel(
        _body,
        out_type=jax.ShapeDtypeStruct((N // ROW, ROW), jnp.float32),
        mesh=plsc.VectorSubcoreMesh(core_axis_name="c", subcore_axis_name="s"),
        scratch_types=[
            pltpu.VMEM_SHARED((LENGTH,), jnp.float32),
            pltpu.VMEM((LOAD_CHUNK,), jnp.float32),
            pltpu.VMEM((2, WROWS, ROW), jnp.int32),
            pltpu.VMEM((2, WROWS, ROW), jnp.float32),
            pltpu.SemaphoreType.DMA,
            pltpu.SemaphoreType.DMA,
            pltpu.SemaphoreType.DMA,
            pltpu.SemaphoreType.DMA,
            pltpu.SemaphoreType.DMA,
        ],
    )(idx2d, u)
    return out.reshape(idx.shape)


# speedup vs baseline: 321.1858x; 1.4658x over previous
import jax
import jax.numpy as jnp
from jax import lax
from jax.experimental import pallas as pl
from jax.experimental.pallas import tpu as pltpu
from jax.experimental.pallas import tpu_sc as plsc

LENGTH = 1_000_000
B, H = 16384, 200
NC, NS = 2, 16
NW = NC * NS
ROWS_PER_W = B // NW
WR = 8
WINDOWS = ROWS_PER_W // WR
LOAD_CHUNK = 50_000
LOAD_CHUNKS = LENGTH // LOAD_CHUNK


def _body(idx_hbm, u_hbm, out_hbm, u_sp, bounce_v, idx_v, out_v,
          sem_g, sem_i0, sem_i1, sem_o0, sem_o1):
    c = lax.axis_index("c")
    s = lax.axis_index("s")
    wid = s * NC + c
    row0 = wid * ROWS_PER_W
    sem_i = (sem_i0, sem_i1)
    sem_o = (sem_o0, sem_o1)

    def idx_rows(wi):
        return idx_hbm.at[pl.ds(row0 + wi * WR, WR)]

    def out_rows(wi):
        return out_hbm.at[pl.ds(row0 + wi * WR, WR)]

    pltpu.async_copy(idx_rows(0), idx_v.at[0], sem_i0)
    pltpu.async_copy(idx_rows(1), idx_v.at[1], sem_i1)

    @pl.loop(s, LOAD_CHUNKS, step=NS)
    def _(ci):
        off = ci * LOAD_CHUNK
        pltpu.sync_copy(u_hbm.at[pl.ds(off, LOAD_CHUNK)], bounce_v)
        pltpu.sync_copy(bounce_v, u_sp.at[pl.ds(off, LOAD_CHUNK)])

    plsc.subcore_barrier()

    @pl.loop(0, WINDOWS, step=2)
    def _(w):
        for b in range(2):
            wi = w + b
            pltpu.make_async_copy(idx_rows(wi), idx_v.at[b], sem_i[b]).wait()
            @pl.when(wi >= 2)
            def _():
                pltpu.make_async_copy(out_v.at[b], out_rows(wi - 2),
                                      sem_o[b]).wait()
            cps = []
            for j in range(WR):
                irow = idx_v.at[b].at[j]
                orow = out_v.at[b].at[j]
                cps.append(pltpu.async_copy(
                    u_sp.at[irow.at[pl.ds(0, 128)]],
                    orow.at[pl.ds(0, 128)], sem_g))
                cps.append(pltpu.async_copy(
                    u_sp.at[irow.at[pl.ds(128, H - 128)]],
                    orow.at[pl.ds(128, H - 128)], sem_g))
            for cp in cps:
                cp.wait()
            @pl.when(wi + 2 < WINDOWS)
            def _():
                pltpu.async_copy(idx_rows(wi + 2), idx_v.at[b], sem_i[b])
            pltpu.async_copy(out_v.at[b], out_rows(wi), sem_o[b])

    pltpu.make_async_copy(out_v.at[0], out_rows(WINDOWS - 2), sem_o0).wait()
    pltpu.make_async_copy(out_v.at[1], out_rows(WINDOWS - 1), sem_o1).wait()


def kernel(idx, u):
    return pl.kernel(
        _body,
        out_type=jax.ShapeDtypeStruct((B, H), jnp.float32),
        mesh=plsc.VectorSubcoreMesh(core_axis_name="c", subcore_axis_name="s"),
        scratch_types=[
            pltpu.VMEM_SHARED((LENGTH,), jnp.float32),
            pltpu.VMEM((LOAD_CHUNK,), jnp.float32),
            pltpu.VMEM((2, WR, H), jnp.int32),
            pltpu.VMEM((2, WR, H), jnp.float32),
            pltpu.SemaphoreType.DMA,
            pltpu.SemaphoreType.DMA,
            pltpu.SemaphoreType.DMA,
            pltpu.SemaphoreType.DMA,
            pltpu.SemaphoreType.DMA,
        ],
    )(idx, u)
